# baseline (speedup 1.0000x reference)
import functools

import jax
import jax.numpy as jnp
from jax import lax
from jax.experimental import pallas as pl
from jax.experimental.pallas import tpu as pltpu

_CP = 128
_C3 = 256


def _fused_kernel(
    x_ref,
    w1_ref, b1_ref,
    w2_ref, b2_ref,
    w3_ref, b3_ref,
    wf1_ref, bf1_ref,
    wf2_ref, bf2_ref,
    policy_ref,
    value_ref,
    *, imgs, H, W, A,
):
    HW = H * W
    R = imgs * HW

    rows = lax.broadcasted_iota(jnp.int32, (R, 1), 0)
    p = rows // imgs
    yy = p // W
    xx = p % W

    valid = {}
    for t in range(9):
        oy, ox = t // 3 - 1, t % 3 - 1
        if oy == 0 and ox == 0:
            continue
        valid[t] = ((yy + oy >= 0) & (yy + oy < H) &
                    (xx + ox >= 0) & (xx + ox < W))

    def im2col_patch(act_bf16):
        taps = []
        for t in range(9):
            oy, ox = t // 3 - 1, t % 3 - 1
            if oy == 0 and ox == 0:
                taps.append(act_bf16)
                continue
            s = oy * W + ox
            shifted = pltpu.roll(act_bf16, shift=(-s * imgs) % R, axis=0)
            taps.append(jnp.where(valid[t], shifted, 0).astype(jnp.bfloat16))
        return jnp.concatenate(taps, axis=1)

    def gemm_bias_relu(lhs_bf16, w_ref, b_ref):
        y = jnp.dot(lhs_bf16, w_ref[...], preferred_element_type=jnp.float32)
        return jnp.maximum(y + b_ref[...], 0.0)

    xb = jnp.transpose(x_ref[...], (1, 0, 2)).reshape(R, _CP)

    wz = jnp.zeros((_CP - 3, _CP), jnp.bfloat16)
    w1e = jnp.concatenate(
        [piece for t in range(9) for piece in (w1_ref[3 * t:3 * t + 3, :], wz)],
        axis=0)

    a1 = jnp.maximum(
        jnp.dot(im2col_patch(xb), w1e, preferred_element_type=jnp.float32)
        + b1_ref[...], 0.0)
    a2 = gemm_bias_relu(im2col_patch(a1.astype(jnp.bfloat16)), w2_ref, b2_ref)
    a3 = gemm_bias_relu(im2col_patch(a2.astype(jnp.bfloat16)), w3_ref, b3_ref)

    a3b = a3.astype(jnp.bfloat16)
    h1 = jnp.zeros((imgs, 512), jnp.float32)
    for q in range(HW):
        h1 = h1 + jnp.dot(a3b[q * imgs:(q + 1) * imgs, :],
                          wf1_ref[pl.ds(q * _C3, _C3), :],
                          preferred_element_type=jnp.float32)
    h1 = jnp.maximum(h1 + bf1_ref[...], 0.0)

    logits = jnp.dot(h1.astype(jnp.bfloat16), wf2_ref[...],
                     preferred_element_type=jnp.float32) + bf2_ref[...]

    col = lax.broadcasted_iota(jnp.int32, logits.shape, 1)
    is_real = col < A
    masked = jnp.where(is_real, logits, -1e30)
    mx = jnp.max(masked, axis=1, keepdims=True)
    e = jnp.exp(masked - mx)
    probs = e / jnp.sum(e, axis=1, keepdims=True)
    policy_ref[...] = probs[:, :A]
    value_ref[...] = jnp.sum(jnp.where(is_real, logits, 0.0),
                             axis=1, keepdims=True)


def kernel(x, conv1_w, conv1_b, conv2_w, conv2_b, conv3_w, conv3_b,
           fc1_w, fc1_b, fc2_w, fc2_b):
    B, Cin, H, W = x.shape
    HW = H * W
    A_pad = fc2_w.shape[1]
    action_size = 82

    IMG = next(g for g in (64, 32, 16, 8, 4, 2, 1) if B % g == 0)
    NBLK = B // IMG

    xg = jnp.pad(x.reshape(B, Cin, HW), ((0, 0), (0, _CP - Cin), (0, 0)))
    xg = jnp.transpose(xg, (0, 2, 1)).astype(jnp.bfloat16)

    weights = (conv1_w, conv1_b, conv2_w, conv2_b, conv3_w, conv3_b,
               fc1_w, fc1_b, fc2_w, fc2_b)

    flops = (2 * B * HW * (9 * _CP * _CP + 9 * _CP * _CP + 9 * _CP * _C3 + _C3 * 512)
             + 2 * B * 512 * A_pad)
    bytes_accessed = (int(xg.size) * 2
                      + sum(int(a.size) * a.dtype.itemsize for a in weights)
                      + B * A_pad * 4 + B * 4)
    cost = pl.CostEstimate(flops=flops, transcendentals=B * A_pad,
                           bytes_accessed=bytes_accessed)

    kernel_fn = functools.partial(_fused_kernel, imgs=IMG, H=H, W=W,
                                  A=action_size)

    def _pinned(a):
        return pl.BlockSpec(a.shape, lambda i: (0,) * a.ndim)

    in_specs = [pl.BlockSpec((IMG, HW, _CP), lambda i: (i, 0, 0))]
    in_specs += [_pinned(a) for a in weights]

    policy, value = pl.pallas_call(
        kernel_fn,
        out_shape=(
            jax.ShapeDtypeStruct((B, action_size), jnp.float32),
            jax.ShapeDtypeStruct((B, 1), jnp.float32),
        ),
        grid=(NBLK,),
        in_specs=in_specs,
        out_specs=(
            pl.BlockSpec((IMG, action_size), lambda i: (i, 0)),
            pl.BlockSpec((IMG, 1), lambda i: (i, 0)),
        ),
        compiler_params=pltpu.CompilerParams(
            dimension_semantics=("parallel",),
            vmem_limit_bytes=56 * 1024 * 1024,
        ),
        cost_estimate=cost,
    )(xg, *weights)
    return policy, value


# speedup vs baseline: 1.1859x; 1.1859x over previous
import functools

import jax
import jax.numpy as jnp
from jax import lax
from jax.experimental import pallas as pl
from jax.experimental.pallas import tpu as pltpu

_CP = 128
_C3 = 256


def _fused_kernel(
    x_ref,
    w1_ref, b1_ref,
    w2_ref, b2_ref,
    w3_ref, b3_ref,
    wf1_ref, bf1_ref,
    wf2_ref, bf2_ref,
    policy_ref,
    value_ref,
    wf1_vmem, wf1_sem,
    *, imgs, H, W, A,
):
    HW = H * W
    R = imgs * HW

    wf1_copy = pltpu.make_async_copy(wf1_ref, wf1_vmem, wf1_sem)

    @pl.when(pl.program_id(0) == 0)
    def _start_wf1():
        wf1_copy.start()

    rows = lax.broadcasted_iota(jnp.int32, (R, 1), 0)
    p = rows // imgs
    yy = p // W
    xx = p % W

    valid = {}
    for t in range(9):
        oy, ox = t // 3 - 1, t % 3 - 1
        if oy == 0 and ox == 0:
            continue
        valid[t] = ((yy + oy >= 0) & (yy + oy < H) &
                    (xx + ox >= 0) & (xx + ox < W))

    def im2col_patch(act_bf16):
        taps = []
        for t in range(9):
            oy, ox = t // 3 - 1, t % 3 - 1
            if oy == 0 and ox == 0:
                taps.append(act_bf16)
                continue
            s = oy * W + ox
            shifted = pltpu.roll(act_bf16, shift=(-s * imgs) % R, axis=0)
            taps.append(jnp.where(valid[t], shifted, 0).astype(jnp.bfloat16))
        return jnp.concatenate(taps, axis=1)

    def gemm_bias_relu(lhs_bf16, w_ref, b_ref):
        y = jnp.dot(lhs_bf16, w_ref[...], preferred_element_type=jnp.float32)
        return jnp.maximum(y + b_ref[...], 0.0)

    a1 = gemm_bias_relu(x_ref[0], w1_ref, b1_ref)
    a2 = gemm_bias_relu(im2col_patch(a1.astype(jnp.bfloat16)), w2_ref, b2_ref)
    a3 = gemm_bias_relu(im2col_patch(a2.astype(jnp.bfloat16)), w3_ref, b3_ref)

    @pl.when(pl.program_id(0) == 0)
    def _wait_wf1():
        wf1_copy.wait()

    a3b = a3.astype(jnp.bfloat16)
    h1 = jnp.zeros((imgs, 512), jnp.float32)
    for q in range(HW):
        h1 = h1 + jnp.dot(a3b[q * imgs:(q + 1) * imgs, :],
                          wf1_vmem[pl.ds(q * _C3, _C3), :],
                          preferred_element_type=jnp.float32)
    h1 = jnp.maximum(h1 + bf1_ref[...], 0.0)

    logits = jnp.dot(h1.astype(jnp.bfloat16), wf2_ref[...],
                     preferred_element_type=jnp.float32) + bf2_ref[...]

    col = lax.broadcasted_iota(jnp.int32, logits.shape, 1)
    is_real = col < A
    masked = jnp.where(is_real, logits, -1e30)
    mx = jnp.max(masked, axis=1, keepdims=True)
    e = jnp.exp(masked - mx)
    probs = e / jnp.sum(e, axis=1, keepdims=True)
    policy_ref[...] = probs[:, :A]
    value_ref[...] = jnp.sum(jnp.where(is_real, logits, 0.0),
                             axis=1, keepdims=True)


def kernel(x, conv1_w, conv1_b, conv2_w, conv2_b, conv3_w, conv3_b,
           fc1_w, fc1_b, fc2_w, fc2_b):
    B, Cin, H, W = x.shape
    HW = H * W
    A_pad = fc2_w.shape[1]
    action_size = 82

    IMG = next(g for g in (64, 32, 16, 8, 4, 2, 1) if B % g == 0)
    NBLK = B // IMG

    x_nhwc = jnp.transpose(x, (0, 2, 3, 1))
    xp = jnp.pad(x_nhwc, ((0, 0), (1, 1), (1, 1), (0, 0)))
    taps = [xp[:, dy:dy + H, dx:dx + W, :] for dy in range(3) for dx in range(3)]
    patches = jnp.stack(taps, axis=3).reshape(B, HW, 9 * Cin)
    patches = jnp.pad(patches, ((0, 0), (0, 0), (0, _CP - 9 * Cin)))
    xg = (patches.astype(jnp.bfloat16)
          .reshape(NBLK, IMG, HW, _CP)
          .transpose(0, 2, 1, 3)
          .reshape(NBLK, HW * IMG, _CP))

    weights = (conv1_w, conv1_b, conv2_w, conv2_b, conv3_w, conv3_b,
               fc1_w, fc1_b, fc2_w, fc2_b)

    flops = (2 * B * HW * (_CP * _CP + 9 * _CP * _CP + 9 * _CP * _C3 + _C3 * 512)
             + 2 * B * 512 * A_pad)
    bytes_accessed = (int(xg.size) * 2
                      + sum(int(a.size) * a.dtype.itemsize for a in weights)
                      + B * A_pad * 4 + B * 4)
    cost = pl.CostEstimate(flops=flops, transcendentals=B * A_pad,
                           bytes_accessed=bytes_accessed)

    kernel_fn = functools.partial(_fused_kernel, imgs=IMG, H=H, W=W,
                                  A=action_size)

    def _pinned(a):
        return pl.BlockSpec(a.shape, lambda i: (0,) * a.ndim)

    in_specs = [pl.BlockSpec((1, HW * IMG, _CP), lambda i: (i, 0, 0))]
    in_specs += [_pinned(a) for a in weights]
    in_specs[7] = pl.BlockSpec(memory_space=pl.ANY)

    policy, value = pl.pallas_call(
        kernel_fn,
        out_shape=(
            jax.ShapeDtypeStruct((B, action_size), jnp.float32),
            jax.ShapeDtypeStruct((B, 1), jnp.float32),
        ),
        grid=(NBLK,),
        in_specs=in_specs,
        out_specs=(
            pl.BlockSpec((IMG, action_size), lambda i: (i, 0)),
            pl.BlockSpec((IMG, 1), lambda i: (i, 0)),
        ),
        scratch_shapes=[
            pltpu.VMEM(fc1_w.shape, fc1_w.dtype),
            pltpu.SemaphoreType.DMA,
        ],
        compiler_params=pltpu.CompilerParams(
            dimension_semantics=("arbitrary",),
            vmem_limit_bytes=56 * 1024 * 1024,
        ),
        cost_estimate=cost,
    )(xg, *weights)
    return policy, value
